# baseline (speedup 1.0000x reference)
import functools

import jax
import jax.numpy as jnp
from jax.experimental import pallas as pl
from jax.experimental.pallas import tpu as pltpu

_K = 32
_GF_DIMS = [(3, 16), (16, 16), (16, 16), (16, 32), (32, 256)]


def _gn_cn(x, gamma_col, beta_col):
    C = x.shape[0]
    C4 = C // 4
    blocks = []
    for g in range(4):
        blk = x[g * C4:(g + 1) * C4, :]
        m = jnp.mean(blk)
        v = jnp.mean((blk - m) ** 2)
        blocks.append((blk - m) / jnp.sqrt(v + 1e-5))
    xn = jnp.concatenate(blocks, axis=0)
    return xn * gamma_col + beta_col


def _impl(kf_t_ref, km_ref, km_t_ref,
          g0w, g0b, g0g, g0e, g1w, g1b, g1g, g1e, g2w, g2b, g2g, g2e,
          g3w, g3b, g3g, g3e, g4w, g4b, g4g, g4e,
          wkf, wc, wgf, wgm, d0b, d0g, d0e,
          d1w, d1b, d1g, d1e, d2w, d2b,
          out_ref, dist_ref, cand_ref, z_ref, disp_ref):
    f32 = jnp.float32
    kf_t = kf_t_ref[0]
    km = km_ref[0]
    km_t = km_t_ref[0]
    N = kf_t.shape[1]
    M = km.shape[0]

    def dot(a, b):
        return jnp.dot(a, b, preferred_element_type=f32)

    gfw = [(g0w, g0b, g0g, g0e), (g1w, g1b, g1g, g1e), (g2w, g2b, g2g, g2e),
           (g3w, g3b, g3g, g3e), (g4w, g4b, g4g, g4e)]

    def gf_forward(x):
        for (w, b, g, e) in gfw:
            x = dot(w[...], x) + b[...]
            x = jnp.maximum(_gn_cn(x, g[...], e[...]), 0.0)
        return jnp.max(x, axis=1, keepdims=True)

    gfix = gf_forward(kf_t)
    gmov = gf_forward(km_t)

    base_vec = dot(wgf[...], gfix) + dot(wgm[...], gmov) + d0b[...]
    base = dot(wkf[...], kf_t) + base_vec

    d = ((km[:, 0:1] - kf_t[0:1, :]) ** 2
         + (km[:, 1:2] - kf_t[1:2, :]) ** 2
         + (km[:, 2:3] - kf_t[2:3, :]) ** 2)
    dist_ref[...] = d

    iota0 = jax.lax.broadcasted_iota(jnp.int32, (M, N), 0)

    def knn_body(k, _):
        dd = dist_ref[...]
        mv = jnp.min(dd, axis=0, keepdims=True)
        idx = jnp.min(jnp.where(dd <= mv, iota0, M), axis=0,
                      keepdims=True)
        onehot = iota0 == idx
        oh = jnp.where(onehot, 1.0, 0.0)
        gath = dot(km_t, oh)
        cand_ref[pl.ds(k, 1)] = (gath - kf_t)[None]
        dist_ref[...] = jnp.where(onehot, jnp.inf, dd)
        return 0

    jax.lax.fori_loop(0, _K, knn_body, 0, unroll=32)

    wc_v = wc[...]
    npix = f32(64 * _K * N)

    def gn_affine(sums, sumsqs, n, gamma_col, beta_col, C4):
        means = [s / n for s in sums]
        variances = [jnp.maximum(q / n - m * m, 0.0)
                     for q, m in zip(sumsqs, means)]
        mcol = jnp.concatenate(
            [jnp.zeros((C4, 1), f32) + m for m in means], axis=0)
        vcol = jnp.concatenate(
            [jnp.zeros((C4, 1), f32) + v for v in variances], axis=0)
        s = gamma_col / jnp.sqrt(vcol + 1e-5)
        t = beta_col - mcol * s
        return s, t

    cand_all0 = cand_ref[...]
    candsum = jnp.sum(cand_all0, axis=0)
    scand = jnp.sum(candsum, axis=1, keepdims=True)
    cj = [cand_all0[:, j, :] for j in range(3)]
    G = {}
    for j in range(3):
        for l in range(j, 3):
            G[(j, l)] = jnp.sum(cj[j] * cj[l])
    w3 = [wc_v[:, j:j + 1] for j in range(3)]
    q = (w3[0] * w3[0] * G[(0, 0)] + w3[1] * w3[1] * G[(1, 1)]
         + w3[2] * w3[2] * G[(2, 2)]
         + 2.0 * (w3[0] * w3[1] * G[(0, 1)] + w3[0] * w3[2] * G[(0, 2)]
                  + w3[1] * w3[2] * G[(1, 2)]))
    alin = dot(wc_v, candsum)
    crosscol = jnp.sum(base * alin, axis=1, keepdims=True)
    lin1 = dot(wc_v, scand)
    sbcol = jnp.sum(base, axis=1, keepdims=True)
    sb2col = jnp.sum(base * base, axis=1, keepdims=True)
    kf32 = f32(_K)
    sumcol = lin1 + kf32 * sbcol
    sqcol = q + 2.0 * crosscol + kf32 * sb2col
    s0 = [jnp.sum(sumcol[g * 64:(g + 1) * 64, :]) for g in range(4)]
    q0 = [jnp.sum(sqcol[g * 64:(g + 1) * 64, :]) for g in range(4)]
    s0c, t0c = gn_affine(s0, q0, npix, d0g[...], d0e[...], 64)

    wc_s = wc_v * s0c
    base_s = base * s0c + t0c

    d1w_v = d1w[...]
    d1b_v = d1b[...]
    npix1 = f32(32 * _K * N)

    def p2(k, s):
        ck = cand_ref[pl.ds(k, 1)][0]
        y = jnp.maximum(dot(wc_s, ck) + base_s, 0.0)
        z = dot(d1w_v, y) + d1b_v
        z_ref[pl.ds(k, 1)] = z[None]
        blks = [z[g * 32:(g + 1) * 32, :] for g in range(4)]
        return (tuple(s[0][g] + jnp.sum(blks[g]) for g in range(4)),
                tuple(s[1][g] + jnp.sum(blks[g] * blks[g])
                      for g in range(4)))

    s1, q1 = jax.lax.fori_loop(0, _K, p2, ((f32(0),) * 4, (f32(0),) * 4), unroll=32)
    s1c, t1c = gn_affine(s1, q1, npix1, d1g[...], d1e[...], 32)

    d2w_v = d2w[...]
    d2b_v = d2b[...]

    def p3(k, _):
        z = z_ref[pl.ds(k, 1)][0]
        y1 = jnp.maximum(z * s1c + t1c, 0.0)
        disp_ref[pl.ds(k, 1)] = dot(d2w_v, y1) + d2b_v
        return 0

    jax.lax.fori_loop(0, _K, p3, 0, unroll=32)

    dsp = disp_ref[...]
    mx = jnp.max(dsp, axis=0, keepdims=True)
    e = jnp.exp(dsp - mx)
    w = e / jnp.sum(e, axis=0, keepdims=True)
    cand_all = cand_ref[...]
    out_ref[0] = jnp.sum(cand_all * w[:, None, :], axis=0)


def kernel(kpts_fixed, kpts_moving,
           gf0_w, gf0_b, gf0_g, gf0_be, gf1_w, gf1_b, gf1_g, gf1_be,
           gf2_w, gf2_b, gf2_g, gf2_be, gf3_w, gf3_b, gf3_g, gf3_be,
           gf4_w, gf4_b, gf4_g, gf4_be,
           dp0_w, dp0_b, dp0_g, dp0_be, dp1_w, dp1_b, dp1_g, dp1_be,
           dp2_w, dp2_b):
    f32 = jnp.float32
    B, N, _ = kpts_fixed.shape
    M = kpts_moving.shape[1]
    kf_t = jnp.transpose(kpts_fixed, (0, 2, 1))
    km_t = jnp.transpose(kpts_moving, (0, 2, 1))

    col = lambda v: v.reshape(-1, 1)
    wkf = dp0_w[:, 0:3]
    wc = dp0_w[:, 3:6]
    wgf = dp0_w[:, 6:262]
    wgm = dp0_w[:, 262:518]

    gf_args = []
    for (w, b, g, e) in [(gf0_w, gf0_b, gf0_g, gf0_be),
                         (gf1_w, gf1_b, gf1_g, gf1_be),
                         (gf2_w, gf2_b, gf2_g, gf2_be),
                         (gf3_w, gf3_b, gf3_g, gf3_be),
                         (gf4_w, gf4_b, gf4_g, gf4_be)]:
        gf_args += [w, col(b), col(g), col(e)]

    args = ([kf_t, kpts_moving, km_t] + gf_args +
            [wkf, wc, wgf, wgm, col(dp0_b), col(dp0_g), col(dp0_be),
             dp1_w, col(dp1_b), col(dp1_g), col(dp1_be),
             dp2_w, dp2_b.reshape(1, 1)])

    def full_spec(a):
        shp = a.shape
        return pl.BlockSpec(shp, lambda b, _n=len(shp): (0,) * _n)

    in_specs = ([pl.BlockSpec((1, 3, N), lambda b: (b, 0, 0)),
                 pl.BlockSpec((1, M, 3), lambda b: (b, 0, 0)),
                 pl.BlockSpec((1, 3, M), lambda b: (b, 0, 0))] +
                [full_spec(a) for a in args[3:]])

    out_t = pl.pallas_call(
        _impl,
        grid=(B,),
        in_specs=in_specs,
        out_specs=pl.BlockSpec((1, 3, N), lambda b: (b, 0, 0)),
        out_shape=jax.ShapeDtypeStruct((B, 3, N), f32),
        scratch_shapes=[
            pltpu.VMEM((M, N), f32),
            pltpu.VMEM((_K, 3, N), f32),
            pltpu.VMEM((_K, 128, N), f32),
            pltpu.VMEM((_K, N), f32),
        ],
    )(*args)
    return jnp.transpose(out_t, (0, 2, 1))


# speedup vs baseline: 1.2195x; 1.2195x over previous
import jax
import jax.numpy as jnp
from jax.experimental import pallas as pl
from jax.experimental.pallas import tpu as pltpu

_K = 32


def _gn_cn(x, gamma_col, beta_col):
    C = x.shape[0]
    C4 = C // 4
    blocks = []
    for g in range(4):
        blk = x[g * C4:(g + 1) * C4, :]
        m = jnp.mean(blk)
        v = jnp.mean((blk - m) ** 2)
        blocks.append((blk - m) / jnp.sqrt(v + 1e-5))
    xn = jnp.concatenate(blocks, axis=0)
    return xn * gamma_col + beta_col


def _impl(kf_t_ref, km_ref, km_t_ref,
          g0w, g0b, g0g, g0e, g1w, g1b, g1g, g1e, g2w, g2b, g2g, g2e,
          g3w, g3b, g3g, g3e, g4w, g4b, g4g, g4e,
          wkf, wc, wgf, wgm, d0b, d0g, d0e,
          d1w, d1b, d1g, d1e, d2w, d2b,
          out_ref, dist_ref, cand_ref, z_ref, disp_ref):
    f32 = jnp.float32
    kf_t = kf_t_ref[0]
    km = km_ref[0]
    km_t = km_t_ref[0]
    N = kf_t.shape[1]
    M = km.shape[0]

    def dot(a, b):
        return jnp.dot(a, b, preferred_element_type=f32)

    gfw = [(g0w, g0b, g0g, g0e), (g1w, g1b, g1g, g1e), (g2w, g2b, g2g, g2e),
           (g3w, g3b, g3g, g3e), (g4w, g4b, g4g, g4e)]

    def gf_forward(x):
        for (w, b, g, e) in gfw:
            x = dot(w[...], x) + b[...]
            x = jnp.maximum(_gn_cn(x, g[...], e[...]), 0.0)
        return jnp.max(x, axis=1, keepdims=True)

    gfix = gf_forward(kf_t)
    gmov = gf_forward(km_t)

    base_vec = dot(wgf[...], gfix) + dot(wgm[...], gmov) + d0b[...]
    base = dot(wkf[...], kf_t) + base_vec

    d = ((km[:, 0:1] - kf_t[0:1, :]) ** 2
         + (km[:, 1:2] - kf_t[1:2, :]) ** 2
         + (km[:, 2:3] - kf_t[2:3, :]) ** 2)
    dist_ref[...] = d

    iota0 = jax.lax.broadcasted_iota(jnp.int32, (M, N), 0)

    def knn_body(k, _):
        dd = dist_ref[...]
        mv = jnp.min(dd, axis=0, keepdims=True)
        idx = jnp.min(jnp.where(dd <= mv, iota0, M), axis=0,
                      keepdims=True)
        onehot = iota0 == idx
        oh = jnp.where(onehot, 1.0, 0.0)
        gath = dot(km_t, oh)
        cand_ref[pl.ds(k, 1)] = (gath - kf_t)[None]
        dist_ref[...] = jnp.where(onehot, jnp.inf, dd)
        return 0

    jax.lax.fori_loop(0, _K, knn_body, 0, unroll=16)

    wc_v = wc[...]
    npix = f32(64 * _K * N)

    def gn_affine(sums, sumsqs, n, gamma_col, beta_col, C4):
        means = [s / n for s in sums]
        variances = [jnp.maximum(q / n - m * m, 0.0)
                     for q, m in zip(sumsqs, means)]
        mcol = jnp.concatenate(
            [jnp.zeros((C4, 1), f32) + m for m in means], axis=0)
        vcol = jnp.concatenate(
            [jnp.zeros((C4, 1), f32) + v for v in variances], axis=0)
        s = gamma_col / jnp.sqrt(vcol + 1e-5)
        t = beta_col - mcol * s
        return s, t

    cand_all0 = cand_ref[...]
    candsum = jnp.sum(cand_all0, axis=0)
    scand = jnp.sum(candsum, axis=1, keepdims=True)
    cj = [cand_all0[:, j, :] for j in range(3)]
    G = {}
    for j in range(3):
        for l in range(j, 3):
            G[(j, l)] = jnp.sum(cj[j] * cj[l])
    w3 = [wc_v[:, j:j + 1] for j in range(3)]
    q = (w3[0] * w3[0] * G[(0, 0)] + w3[1] * w3[1] * G[(1, 1)]
         + w3[2] * w3[2] * G[(2, 2)]
         + 2.0 * (w3[0] * w3[1] * G[(0, 1)] + w3[0] * w3[2] * G[(0, 2)]
                  + w3[1] * w3[2] * G[(1, 2)]))
    alin = dot(wc_v, candsum)
    crosscol = jnp.sum(base * alin, axis=1, keepdims=True)
    lin1 = dot(wc_v, scand)
    sbcol = jnp.sum(base, axis=1, keepdims=True)
    sb2col = jnp.sum(base * base, axis=1, keepdims=True)
    kf32 = f32(_K)
    sumcol = lin1 + kf32 * sbcol
    sqcol = q + 2.0 * crosscol + kf32 * sb2col
    s0 = [jnp.sum(sumcol[g * 64:(g + 1) * 64, :]) for g in range(4)]
    q0 = [jnp.sum(sqcol[g * 64:(g + 1) * 64, :]) for g in range(4)]
    s0c, t0c = gn_affine(s0, q0, npix, d0g[...], d0e[...], 64)

    wc_s = wc_v * s0c
    base_s = base * s0c + t0c

    d1w_v = d1w[...]
    d1b_v = d1b[...]
    npix1 = f32(32 * _K * N)

    def p2(k, s):
        ck = cand_ref[pl.ds(k, 1)][0]
        y = jnp.maximum(dot(wc_s, ck) + base_s, 0.0)
        z = dot(d1w_v, y) + d1b_v
        z_ref[pl.ds(k, 1)] = z[None]
        blks = [z[g * 32:(g + 1) * 32, :] for g in range(4)]
        return (tuple(s[0][g] + jnp.sum(blks[g]) for g in range(4)),
                tuple(s[1][g] + jnp.sum(blks[g] * blks[g])
                      for g in range(4)))

    s1, q1 = jax.lax.fori_loop(0, _K, p2, ((f32(0),) * 4, (f32(0),) * 4), unroll=16)
    s1c, t1c = gn_affine(s1, q1, npix1, d1g[...], d1e[...], 32)

    d2w_v = d2w[...]
    d2b_v = d2b[...]

    def p3(k, _):
        z = z_ref[pl.ds(k, 1)][0]
        y1 = jnp.maximum(z * s1c + t1c, 0.0)
        disp_ref[pl.ds(k, 1)] = dot(d2w_v, y1) + d2b_v
        return 0

    jax.lax.fori_loop(0, _K, p3, 0, unroll=16)

    dsp = disp_ref[...]
    mx = jnp.max(dsp, axis=0, keepdims=True)
    e = jnp.exp(dsp - mx)
    w = e / jnp.sum(e, axis=0, keepdims=True)
    cand_all = cand_ref[...]
    out_ref[0] = jnp.sum(cand_all * w[:, None, :], axis=0)


def kernel(kpts_fixed, kpts_moving,
           gf0_w, gf0_b, gf0_g, gf0_be, gf1_w, gf1_b, gf1_g, gf1_be,
           gf2_w, gf2_b, gf2_g, gf2_be, gf3_w, gf3_b, gf3_g, gf3_be,
           gf4_w, gf4_b, gf4_g, gf4_be,
           dp0_w, dp0_b, dp0_g, dp0_be, dp1_w, dp1_b, dp1_g, dp1_be,
           dp2_w, dp2_b):
    f32 = jnp.float32
    B, N, _ = kpts_fixed.shape
    M = kpts_moving.shape[1]
    kf_t = jnp.transpose(kpts_fixed, (0, 2, 1))
    km_t = jnp.transpose(kpts_moving, (0, 2, 1))

    col = lambda v: v.reshape(-1, 1)
    wkf = dp0_w[:, 0:3]
    wc = dp0_w[:, 3:6]
    wgf = dp0_w[:, 6:262]
    wgm = dp0_w[:, 262:518]

    gf_args = []
    for (w, b, g, e) in [(gf0_w, gf0_b, gf0_g, gf0_be),
                         (gf1_w, gf1_b, gf1_g, gf1_be),
                         (gf2_w, gf2_b, gf2_g, gf2_be),
                         (gf3_w, gf3_b, gf3_g, gf3_be),
                         (gf4_w, gf4_b, gf4_g, gf4_be)]:
        gf_args += [w, col(b), col(g), col(e)]

    args = ([kf_t, kpts_moving, km_t] + gf_args +
            [wkf, wc, wgf, wgm, col(dp0_b), col(dp0_g), col(dp0_be),
             dp1_w, col(dp1_b), col(dp1_g), col(dp1_be),
             dp2_w, dp2_b.reshape(1, 1)])

    def full_spec(a):
        shp = a.shape
        return pl.BlockSpec(shp, lambda b, _n=len(shp): (0,) * _n)

    in_specs = ([pl.BlockSpec((1, 3, N), lambda b: (b, 0, 0)),
                 pl.BlockSpec((1, M, 3), lambda b: (b, 0, 0)),
                 pl.BlockSpec((1, 3, M), lambda b: (b, 0, 0))] +
                [full_spec(a) for a in args[3:]])

    out_t = pl.pallas_call(
        _impl,
        grid=(B,),
        in_specs=in_specs,
        out_specs=pl.BlockSpec((1, 3, N), lambda b: (b, 0, 0)),
        out_shape=jax.ShapeDtypeStruct((B, 3, N), f32),
        scratch_shapes=[
            pltpu.VMEM((M, N), f32),
            pltpu.VMEM((_K, 3, N), f32),
            pltpu.VMEM((_K, 128, N), f32),
            pltpu.VMEM((_K, N), f32),
        ],
    )(*args)
    return jnp.transpose(out_t, (0, 2, 1))
